# baseline (speedup 1.0000x reference)
import functools

import jax
import jax.numpy as jnp
from jax import lax
from jax.experimental import pallas as pl
from jax.experimental.pallas import tpu as pltpu
from jax.experimental.pallas import tpu_sc as plsc

_NC = 2
_NS = 16
_NW = _NC * _NS
_CHUNK = 40


@functools.lru_cache(maxsize=None)
def _build(R, V, dtype_name):
    dtype = jnp.dtype(dtype_name)
    RPW = R // _NW
    NCH = RPW // _CHUNK

    mesh = plsc.VectorSubcoreMesh(core_axis_name="c", subcore_axis_name="s")

    @functools.partial(
        pl.kernel,
        mesh=mesh,
        compiler_params=pltpu.CompilerParams(use_tc_tiling_on_sc=False),
        out_type=jax.ShapeDtypeStruct((R, V), dtype),
        scratch_types=[
            pltpu.VMEM((RPW,), jnp.int32),
            pltpu.VMEM((_CHUNK, V), dtype),
            pltpu.VMEM((_CHUNK, V), dtype),
            pltpu.SemaphoreType.DMA,
            pltpu.SemaphoreType.DMA,
        ],
    )
    def bigram_gather(table_hbm, src_hbm, out_hbm,
                      idx_v, buf0, buf1, sem0, sem1):
        wid = lax.axis_index("s") * _NC + lax.axis_index("c")
        base_row = wid * RPW
        bufs = (buf0, buf1)
        sems = (sem0, sem1)

        pltpu.sync_copy(src_hbm.at[pl.ds(base_row, RPW)], idx_v)

        def gather(c, k):
            return pltpu.make_async_copy(
                table_hbm.at[idx_v.at[pl.ds(c * _CHUNK, _CHUNK)]],
                bufs[k], sems[k])

        gather(0, 0).start()

        def body(g, carry):
            for k in (0, 1):
                c = 2 * g + k

                @pl.when(c + 1 < NCH)
                def _():
                    gather(c + 1, 1 - k).start()

                gather(c, k).wait()
                pltpu.sync_copy(bufs[k],
                                out_hbm.at[pl.ds(base_row + c * _CHUNK,
                                                 _CHUNK)])
            return carry

        lax.fori_loop(0, NCH // 2, body, 0)

    return bigram_gather


def kernel(idx, log_probs):
    B, T = idx.shape
    V = log_probs.shape[1]
    Vr = log_probs.shape[0]
    table = jnp.concatenate(
        [log_probs, jnp.zeros((1, V), log_probs.dtype)], axis=0)
    src = jnp.concatenate(
        [jnp.full((B, 1), Vr, jnp.int32), idx[:, :-1].astype(jnp.int32)],
        axis=1).reshape(B * T)
    out_flat = _build(B * T, V, log_probs.dtype.name)(table, src)
    return out_flat.reshape(B, T, V)


# speedup vs baseline: 1.0023x; 1.0023x over previous
import functools

import jax
import jax.numpy as jnp
from jax import lax
from jax.experimental import pallas as pl
from jax.experimental.pallas import tpu as pltpu
from jax.experimental.pallas import tpu_sc as plsc

_NC = 2
_NS = 16
_NW = _NC * _NS
_CHUNK = 40


@functools.lru_cache(maxsize=None)
def _build(R, V, Vp, Rt, dtype_name):
    dtype = jnp.dtype(dtype_name)
    RPW = R // _NW
    NCH = RPW // _CHUNK

    mesh = plsc.VectorSubcoreMesh(core_axis_name="c", subcore_axis_name="s")

    @functools.partial(
        pl.kernel,
        mesh=mesh,
        out_type=jax.ShapeDtypeStruct((R, Vp), dtype),
        scratch_types=[
            pltpu.VMEM((RPW,), jnp.int32),
            pltpu.VMEM((_CHUNK, Vp), dtype),
            pltpu.VMEM((_CHUNK, Vp), dtype),
            pltpu.SemaphoreType.DMA,
            pltpu.SemaphoreType.DMA,
        ],
    )
    def bigram_gather(table_hbm, src_hbm, out_hbm,
                      idx_v, buf0, buf1, sem0, sem1):
        wid = lax.axis_index("s") * _NC + lax.axis_index("c")
        base_row = wid * RPW
        bufs = (buf0, buf1)
        sems = (sem0, sem1)

        pltpu.sync_copy(src_hbm.at[pl.ds(base_row, RPW)], idx_v)

        def gather(c, k):
            return pltpu.make_async_copy(
                table_hbm.at[idx_v.at[pl.ds(c * _CHUNK, _CHUNK)]],
                bufs[k], sems[k])

        gather(0, 0).start()

        def body(g, carry):
            for k in (0, 1):
                c = 2 * g + k

                @pl.when(c + 1 < NCH)
                def _():
                    gather(c + 1, 1 - k).start()

                gather(c, k).wait()
                pltpu.sync_copy(bufs[k],
                                out_hbm.at[pl.ds(base_row + c * _CHUNK,
                                                 _CHUNK)])
            return carry

        lax.fori_loop(0, NCH // 2, body, 0)

    return bigram_gather


def kernel(idx, log_probs):
    B, T = idx.shape
    Vr, V = log_probs.shape
    Vp = ((V + 127) // 128) * 128
    Rt = ((Vr + 1 + 7) // 8) * 8
    table = jnp.pad(log_probs, ((0, Rt - Vr), (0, Vp - V)))
    src = jnp.concatenate(
        [jnp.full((B, 1), Vr, jnp.int32), idx[:, :-1].astype(jnp.int32)],
        axis=1).reshape(B * T)
    mid = _build(B * T, V, Vp, Rt, log_probs.dtype.name)(table, src)
    return mid[:, :V].reshape(B, T, V)


# speedup vs baseline: 1.0094x; 1.0071x over previous
import functools

import jax
import jax.numpy as jnp
from jax import lax
from jax.experimental import pallas as pl
from jax.experimental.pallas import tpu as pltpu
from jax.experimental.pallas import tpu_sc as plsc

_NC = 2
_NS = 16
_NW = _NC * _NS
_CHUNK = 40
_NBUF = 3


@functools.lru_cache(maxsize=None)
def _build(R, V, dtype_name):
    dtype = jnp.dtype(dtype_name)
    RPW = R // _NW
    NCH = RPW // _CHUNK

    mesh = plsc.VectorSubcoreMesh(core_axis_name="c", subcore_axis_name="s")

    @functools.partial(
        pl.kernel,
        mesh=mesh,
        compiler_params=pltpu.CompilerParams(use_tc_tiling_on_sc=False),
        out_type=jax.ShapeDtypeStruct((R, V), dtype),
        scratch_types=[
            pltpu.VMEM((RPW,), jnp.int32),
            [pltpu.VMEM((_CHUNK, V), dtype) for _ in range(_NBUF)],
            [pltpu.SemaphoreType.DMA for _ in range(_NBUF)],
            [pltpu.SemaphoreType.DMA for _ in range(_NBUF)],
        ],
    )
    def bigram_gather(table_hbm, src_hbm, out_hbm, idx_v, bufs, gsems, ssems):
        wid = lax.axis_index("s") * _NC + lax.axis_index("c")
        base_row = wid * RPW

        pltpu.sync_copy(src_hbm.at[pl.ds(base_row, RPW)], idx_v)

        def gather(c):
            k = c % _NBUF
            return pltpu.make_async_copy(
                table_hbm.at[idx_v.at[pl.ds(c * _CHUNK, _CHUNK)]],
                bufs[k], gsems[k])

        def scatter(c):
            k = c % _NBUF
            return pltpu.make_async_copy(
                bufs[k],
                out_hbm.at[pl.ds(base_row + c * _CHUNK, _CHUNK)],
                ssems[k])

        for c in range(min(_NBUF, NCH)):
            gather(c).start()
        for c in range(NCH):
            gather(c).wait()
            scatter(c).start()
            if c + _NBUF < NCH:
                scatter(c).wait()
                gather(c + _NBUF).start()
        for c in range(max(NCH - _NBUF, 0), NCH):
            scatter(c).wait()

    return bigram_gather


def kernel(idx, log_probs):
    B, T = idx.shape
    V = log_probs.shape[1]
    Vr = log_probs.shape[0]
    table = jnp.concatenate(
        [log_probs, jnp.zeros((1, V), log_probs.dtype)], axis=0)
    src = jnp.concatenate(
        [jnp.full((B, 1), Vr, jnp.int32), idx[:, :-1].astype(jnp.int32)],
        axis=1).reshape(B * T)
    out_flat = _build(B * T, V, log_probs.dtype.name)(table, src)
    return out_flat.reshape(B, T, V)
